# baseline (speedup 1.0000x reference)
import jax
import jax.numpy as jnp
from jax.experimental import pallas as pl
from jax.experimental.pallas import tpu as pltpu


def _fused_kernel(xc_ref, pool_avg_ref, pool_t_ref, wbig_ref, mbias_ref,
                  wcomb_ref, bcomb_ref, xo_ref, po_ref):
    xc = xc_ref[...]

    mean_s = jnp.dot(xc, pool_avg_ref[...], preferred_element_type=jnp.float32)
    mean = jnp.dot(mean_s, pool_t_ref[...], preferred_element_type=jnp.float32)
    d = xc - mean
    var_s = jnp.dot(d * d, pool_avg_ref[...], preferred_element_type=jnp.float32)
    rstd = jnp.dot(jax.lax.rsqrt(var_s + 1e-5), pool_t_ref[...],
                   preferred_element_type=jnp.float32)

    nb = (d * rstd).astype(jnp.bfloat16)
    xm = (jnp.dot(nb, wbig_ref[...], preferred_element_type=jnp.float32)
          + mbias_ref[...])
    xo_ref[...] = xm

    po_ref[...] = (jnp.dot(xm.astype(jnp.bfloat16), wcomb_ref[...],
                           preferred_element_type=jnp.float32)
                   + bcomb_ref[...])


def kernel(x, b_rep_node, Pm, pool_avg, pool_t, wbig_g, mbias, w_comb, b_comb):
    del b_rep_node, Pm
    B, T, seg, D = x.shape
    R = B * T
    SWD = seg * D
    S = pool_avg.shape[1]
    SD_pad = wbig_g.shape[1]
    OL_pad = w_comb.shape[1]
    SD = SD_pad
    OL = OL_pad
    O = 16
    L = OL // O

    xc = x.reshape(R, SWD)

    RB = 512
    n_blocks = R // RB

    wbig_bf = wbig_g.astype(jnp.bfloat16)
    wcomb_bf = w_comb.astype(jnp.bfloat16)

    x_flat, pred_flat = pl.pallas_call(
        _fused_kernel,
        out_shape=(jax.ShapeDtypeStruct((R, SD_pad), jnp.float32),
                   jax.ShapeDtypeStruct((R, OL_pad), jnp.float32)),
        grid_spec=pltpu.PrefetchScalarGridSpec(
            num_scalar_prefetch=0,
            grid=(n_blocks,),
            in_specs=[
                pl.BlockSpec((RB, SWD), lambda r: (r, 0)),
                pl.BlockSpec((SWD, S), lambda r: (0, 0)),
                pl.BlockSpec((S, SWD), lambda r: (0, 0)),
                pl.BlockSpec((SWD, SD_pad), lambda r: (0, 0)),
                pl.BlockSpec((1, SD_pad), lambda r: (0, 0)),
                pl.BlockSpec((SD_pad, OL_pad), lambda r: (0, 0)),
                pl.BlockSpec((1, OL_pad), lambda r: (0, 0)),
            ],
            out_specs=(pl.BlockSpec((RB, SD_pad), lambda r: (r, 0)),
                       pl.BlockSpec((RB, OL_pad), lambda r: (r, 0))),
        ),
        compiler_params=pltpu.CompilerParams(
            dimension_semantics=("parallel",),
            vmem_limit_bytes=32 * 1024 * 1024),
    )(xc, pool_avg, pool_t, wbig_bf, mbias, wcomb_bf, b_comb)

    x_out = x_flat[:, :SD].reshape(B, T, S, D)
    layer_predict = pred_flat[:, :OL].reshape(B, T, O, L)
    return x_out, layer_predict


# speedup vs baseline: 1.6092x; 1.6092x over previous
import jax
import jax.numpy as jnp
from jax.experimental import pallas as pl
from jax.experimental.pallas import tpu as pltpu


def _fused_kernel(x3_ref, pool_avg_ref, pool_t_ref, wbig_ref, mbias_ref,
                  wcomb_ref, bcomb_ref, xo_ref, po_ref):
    RB, seg, D = x3_ref.shape
    SWD = seg * D
    S = xo_ref.shape[1]
    O = po_ref.shape[1]
    xc = x3_ref[...].reshape(RB, SWD)

    mean_s = jnp.dot(xc, pool_avg_ref[...], preferred_element_type=jnp.float32)
    mean = jnp.dot(mean_s, pool_t_ref[...], preferred_element_type=jnp.float32)
    d = xc - mean
    var_s = jnp.dot(d * d, pool_avg_ref[...], preferred_element_type=jnp.float32)
    rstd = jnp.dot(jax.lax.rsqrt(var_s + 1e-5), pool_t_ref[...],
                   preferred_element_type=jnp.float32)

    nb = (d * rstd).astype(jnp.bfloat16)
    xm = (jnp.dot(nb, wbig_ref[...], preferred_element_type=jnp.float32)
          + mbias_ref[...])
    xo_ref[...] = xm.reshape(RB, S, D)

    po = (jnp.dot(xm.astype(jnp.bfloat16), wcomb_ref[...],
                  preferred_element_type=jnp.float32)
          + bcomb_ref[...])
    po_ref[...] = po.reshape(RB, O, po_ref.shape[2])


def kernel(x, b_rep_node, Pm, pool_avg, pool_t, wbig_g, mbias, w_comb, b_comb):
    del b_rep_node, Pm
    B, T, seg, D = x.shape
    R = B * T
    SWD = seg * D
    S = pool_avg.shape[1]
    SD_pad = wbig_g.shape[1]
    OL_pad = w_comb.shape[1]
    O = 16
    L = OL_pad // O

    x3 = x.reshape(R, seg, D)

    RB = 512
    n_blocks = R // RB

    wbig_bf = wbig_g.astype(jnp.bfloat16)
    wcomb_bf = w_comb.astype(jnp.bfloat16)

    x_out3, pred3 = pl.pallas_call(
        _fused_kernel,
        out_shape=(jax.ShapeDtypeStruct((R, S, D), jnp.float32),
                   jax.ShapeDtypeStruct((R, O, L), jnp.float32)),
        grid_spec=pltpu.PrefetchScalarGridSpec(
            num_scalar_prefetch=0,
            grid=(n_blocks,),
            in_specs=[
                pl.BlockSpec((RB, seg, D), lambda r: (r, 0, 0)),
                pl.BlockSpec((SWD, S), lambda r: (0, 0)),
                pl.BlockSpec((S, SWD), lambda r: (0, 0)),
                pl.BlockSpec((SWD, SD_pad), lambda r: (0, 0)),
                pl.BlockSpec((1, SD_pad), lambda r: (0, 0)),
                pl.BlockSpec((SD_pad, OL_pad), lambda r: (0, 0)),
                pl.BlockSpec((1, OL_pad), lambda r: (0, 0)),
            ],
            out_specs=(pl.BlockSpec((RB, S, D), lambda r: (r, 0, 0)),
                       pl.BlockSpec((RB, O, L), lambda r: (r, 0, 0))),
        ),
        compiler_params=pltpu.CompilerParams(
            dimension_semantics=("parallel",),
            vmem_limit_bytes=32 * 1024 * 1024),
    )(x3, pool_avg, pool_t, wbig_bf, mbias, wcomb_bf, b_comb)

    x_out = x_out3.reshape(B, T, S, D)
    layer_predict = pred3.reshape(B, T, O, L)
    return x_out, layer_predict


# speedup vs baseline: 4.5657x; 2.8372x over previous
import jax
import jax.numpy as jnp
from jax import lax
from jax.experimental import pallas as pl
from jax.experimental.pallas import tpu as pltpu


def _fused_kernel(xt_ref, wbig_ref, mbias_ref, wcomb_ref, bcomb_ref,
                  xo_ref, po_ref):
    n_rows, B = xt_ref.shape
    SWD = wbig_ref.shape[0]
    SD = wbig_ref.shape[1]
    OL = wcomb_ref.shape[1]
    WD = 128
    Tb = n_rows // SWD
    n_groups = n_rows // WD

    xt = xt_ref[...]

    x3 = xt.reshape(n_groups, WD, B)
    mean = jnp.mean(x3, axis=1, keepdims=True)
    d = x3 - mean
    var = jnp.mean(d * d, axis=1, keepdims=True)
    norm = d * lax.rsqrt(var + 1e-5)
    nb = norm.reshape(n_rows, B).astype(jnp.bfloat16)

    for t in range(Tb):
        nb_t = nb[t * SWD:(t + 1) * SWD, :]
        xm_t = lax.dot_general(wbig_ref[...], nb_t,
                               (((0,), (0,)), ((), ())),
                               preferred_element_type=jnp.float32)
        xm_t = xm_t + mbias_ref[...]
        xo_ref[t * SD:(t + 1) * SD, :] = xm_t
        po_t = lax.dot_general(wcomb_ref[...], xm_t.astype(jnp.bfloat16),
                               (((0,), (0,)), ((), ())),
                               preferred_element_type=jnp.float32)
        po_ref[t * OL:(t + 1) * OL, :] = po_t + bcomb_ref[...]


def kernel(x, b_rep_node, Pm, pool_avg, pool_t, wbig_g, mbias, w_comb, b_comb):
    del b_rep_node, Pm, pool_avg, pool_t
    B, T, seg, D = x.shape
    SWD = seg * D
    SD_pad = wbig_g.shape[1]
    OL_pad = w_comb.shape[1]
    S = SD_pad // D
    O = 16
    L = OL_pad // O

    xt = jnp.transpose(x, (1, 2, 3, 0)).reshape(T * SWD, B)

    Tb = 8
    n_blocks = T // Tb

    wbig_bf = wbig_g.astype(jnp.bfloat16)
    wcomb_bf = w_comb.astype(jnp.bfloat16)
    mbias_t = mbias.reshape(SD_pad, 1)
    bcomb_t = b_comb.reshape(OL_pad, 1)

    xo_t, po_t = pl.pallas_call(
        _fused_kernel,
        out_shape=(jax.ShapeDtypeStruct((T * SD_pad, B), jnp.float32),
                   jax.ShapeDtypeStruct((T * OL_pad, B), jnp.float32)),
        grid_spec=pltpu.PrefetchScalarGridSpec(
            num_scalar_prefetch=0,
            grid=(n_blocks,),
            in_specs=[
                pl.BlockSpec((Tb * SWD, B), lambda r: (r, 0)),
                pl.BlockSpec((SWD, SD_pad), lambda r: (0, 0)),
                pl.BlockSpec((SD_pad, 1), lambda r: (0, 0)),
                pl.BlockSpec((SD_pad, OL_pad), lambda r: (0, 0)),
                pl.BlockSpec((OL_pad, 1), lambda r: (0, 0)),
            ],
            out_specs=(pl.BlockSpec((Tb * SD_pad, B), lambda r: (r, 0)),
                       pl.BlockSpec((Tb * OL_pad, B), lambda r: (r, 0))),
        ),
        compiler_params=pltpu.CompilerParams(
            dimension_semantics=("parallel",),
            vmem_limit_bytes=50 * 1024 * 1024),
    )(xt, wbig_bf, mbias_t, wcomb_bf, bcomb_t)

    x_out = jnp.transpose(xo_t.reshape(T, S, D, B), (3, 0, 1, 2))
    layer_predict = jnp.transpose(po_t.reshape(T, O, L, B), (3, 0, 1, 2))
    return x_out, layer_predict


# speedup vs baseline: 5.4329x; 1.1899x over previous
import jax
import jax.numpy as jnp
from jax import lax
from jax.experimental import pallas as pl
from jax.experimental.pallas import tpu as pltpu


def _fused_kernel(xt_ref, wbig_ref, mbias_ref, wcomb_ref, bcomb_ref,
                  xo_ref, po_ref):
    n_rows, B = xt_ref.shape
    SWD = wbig_ref.shape[1]
    SD = wbig_ref.shape[0]
    OL = wcomb_ref.shape[0]
    WD = 128
    Tb = n_rows // SWD
    n_groups = n_rows // WD

    xt = xt_ref[...]

    x3 = xt.reshape(n_groups, WD, B)
    mean = jnp.mean(x3, axis=1, keepdims=True)
    d = x3 - mean
    var = jnp.mean(d * d, axis=1, keepdims=True)
    norm = d * lax.rsqrt(var + 1e-5)
    nb = norm.reshape(n_rows, B).astype(jnp.bfloat16)

    for t in range(Tb):
        nb_t = nb[t * SWD:(t + 1) * SWD, :]
        xm_t = jnp.dot(wbig_ref[...], nb_t,
                       preferred_element_type=jnp.float32)
        xm_t = xm_t + mbias_ref[...]
        xo_ref[t * SD:(t + 1) * SD, :] = xm_t
        po_t = jnp.dot(wcomb_ref[...], xm_t.astype(jnp.bfloat16),
                       preferred_element_type=jnp.float32)
        po_ref[t * OL:(t + 1) * OL, :] = po_t + bcomb_ref[...]


def kernel(x, b_rep_node, Pm, pool_avg, pool_t, wbig_g, mbias, w_comb, b_comb):
    del b_rep_node, Pm, pool_avg, pool_t
    B, T, seg, D = x.shape
    SWD = seg * D
    SD_pad = wbig_g.shape[1]
    OL_pad = w_comb.shape[1]
    S = SD_pad // D
    O = 16
    L = OL_pad // O

    xt = jnp.transpose(x, (1, 2, 3, 0)).reshape(T * SWD, B)

    Tb = 8
    n_blocks = T // Tb

    wbig_bf = wbig_g.T.astype(jnp.bfloat16)
    wcomb_bf = w_comb.T.astype(jnp.bfloat16)
    mbias_t = mbias.reshape(SD_pad, 1)
    bcomb_t = b_comb.reshape(OL_pad, 1)

    xo_t, po_t = pl.pallas_call(
        _fused_kernel,
        out_shape=(jax.ShapeDtypeStruct((T * SD_pad, B), jnp.float32),
                   jax.ShapeDtypeStruct((T * OL_pad, B), jnp.float32)),
        grid_spec=pltpu.PrefetchScalarGridSpec(
            num_scalar_prefetch=0,
            grid=(n_blocks,),
            in_specs=[
                pl.BlockSpec((Tb * SWD, B), lambda r: (r, 0)),
                pl.BlockSpec((SD_pad, SWD), lambda r: (0, 0)),
                pl.BlockSpec((SD_pad, 1), lambda r: (0, 0)),
                pl.BlockSpec((OL_pad, SD_pad), lambda r: (0, 0)),
                pl.BlockSpec((OL_pad, 1), lambda r: (0, 0)),
            ],
            out_specs=(pl.BlockSpec((Tb * SD_pad, B), lambda r: (r, 0)),
                       pl.BlockSpec((Tb * OL_pad, B), lambda r: (r, 0))),
        ),
        compiler_params=pltpu.CompilerParams(
            dimension_semantics=("parallel",),
            vmem_limit_bytes=50 * 1024 * 1024),
    )(xt, wbig_bf, mbias_t, wcomb_bf, bcomb_t)

    x_out = jnp.transpose(xo_t.reshape(T, S, D, B), (3, 0, 1, 2))
    layer_predict = jnp.transpose(po_t.reshape(T, O, L, B), (3, 0, 1, 2))
    return x_out, layer_predict


# speedup vs baseline: 7.6657x; 1.4110x over previous
import jax
import jax.numpy as jnp
from jax import lax
from jax.experimental import pallas as pl
from jax.experimental.pallas import tpu as pltpu


def _fused_kernel(xt_ref, wbig_ref, mbias_ref, wcomb_ref, bcomb_ref,
                  xo_ref, po_ref):
    Tb, SWD, B = xt_ref.shape
    SD = wbig_ref.shape[0]
    OL = wcomb_ref.shape[0]
    WD = 128
    n_groups = Tb * SWD // WD

    xt = xt_ref[...]

    x3 = xt.reshape(n_groups, WD, B)
    s1 = jnp.sum(x3, axis=1, keepdims=True)
    s2 = jnp.sum(x3 * x3, axis=1, keepdims=True)
    mean = s1 * (1.0 / WD)
    var = s2 * (1.0 / WD) - mean * mean
    rstd = lax.rsqrt(var + 1e-5)
    scale = rstd.reshape(Tb, SWD // WD, 1, B)
    shift = (mean * rstd).reshape(Tb, SWD // WD, 1, B)
    x4 = x3.reshape(Tb, SWD // WD, WD, B)
    nb2 = jnp.concatenate(
        [(x4[t] * scale[t] - shift[t]).reshape(SWD, B) for t in range(Tb)],
        axis=1).astype(jnp.bfloat16)

    xm2 = (jnp.dot(wbig_ref[...], nb2, preferred_element_type=jnp.float32)
           + mbias_ref[...])
    po2 = (jnp.dot(wcomb_ref[...], xm2.astype(jnp.bfloat16),
                   preferred_element_type=jnp.float32)
           + bcomb_ref[...])

    for t in range(Tb):
        xo_ref[t, :, :] = xm2[:, t * B:(t + 1) * B]
        po_ref[t, :, :] = po2[:, t * B:(t + 1) * B]


def kernel(x, b_rep_node, Pm, pool_avg, pool_t, wbig_g, mbias, w_comb, b_comb):
    del b_rep_node, Pm, pool_avg, pool_t
    B, T, seg, D = x.shape
    SWD = seg * D
    SD_pad = wbig_g.shape[1]
    OL_pad = w_comb.shape[1]
    S = SD_pad // D
    O = 16
    L = OL_pad // O

    xt = jnp.transpose(x, (1, 2, 3, 0)).reshape(T, SWD, B)

    Tb = 8
    n_blocks = T // Tb

    wbig_bf = wbig_g.T.astype(jnp.bfloat16)
    wcomb_bf = w_comb.T.astype(jnp.bfloat16)
    mbias_t = mbias.reshape(SD_pad, 1)
    bcomb_t = b_comb.reshape(OL_pad, 1)

    xo_t, po_t = pl.pallas_call(
        _fused_kernel,
        out_shape=(jax.ShapeDtypeStruct((T, SD_pad, B), jnp.float32),
                   jax.ShapeDtypeStruct((T, OL_pad, B), jnp.float32)),
        grid_spec=pltpu.PrefetchScalarGridSpec(
            num_scalar_prefetch=0,
            grid=(n_blocks,),
            in_specs=[
                pl.BlockSpec((Tb, SWD, B), lambda r: (r, 0, 0)),
                pl.BlockSpec((SD_pad, SWD), lambda r: (0, 0)),
                pl.BlockSpec((SD_pad, 1), lambda r: (0, 0)),
                pl.BlockSpec((OL_pad, SD_pad), lambda r: (0, 0)),
                pl.BlockSpec((OL_pad, 1), lambda r: (0, 0)),
            ],
            out_specs=(pl.BlockSpec((Tb, SD_pad, B), lambda r: (r, 0, 0)),
                       pl.BlockSpec((Tb, OL_pad, B), lambda r: (r, 0, 0))),
        ),
        compiler_params=pltpu.CompilerParams(
            dimension_semantics=("parallel",),
            vmem_limit_bytes=50 * 1024 * 1024),
    )(xt, wbig_bf, mbias_t, wcomb_bf, bcomb_t)

    x_out = jnp.transpose(xo_t.reshape(T, S, D, B), (3, 0, 1, 2))
    layer_predict = jnp.transpose(po_t.reshape(T, O, L, B), (3, 0, 1, 2))
    return x_out, layer_predict
